# Initial kernel scaffold; baseline (speedup 1.0000x reference)
import jax
import jax.numpy as jnp
from jax.experimental import pallas as pl


def kernel(x):
    raise NotImplementedError("write your pallas kernel here")


# speedup vs baseline: 5.8407x; 5.8407x over previous
import jax
import jax.numpy as jnp
from jax import lax
from jax.experimental import pallas as pl

GROUP = 128
BLOCK_ROWS = 1024


def _quant_body(x_ref, o_ref):
    xb = x_ref[...]
    mn = jnp.min(xb, axis=-1, keepdims=True)
    mx = jnp.max(xb, axis=-1, keepdims=True)
    scale = jnp.maximum((mx - mn) * (1.0 / 24.0), 1e-5)
    r = 1.0 / scale
    zp = -12.0 - mn * r
    t = xb * r + zp
    ui = lax.bitcast_convert_type(t, jnp.uint32)
    sign = ui & jnp.uint32(0x80000000)
    a = jnp.minimum(jnp.abs(t), 12.0)
    ab = lax.bitcast_convert_type(a, jnp.uint32)
    bt = lax.bitcast_convert_type(
        (ab + jnp.uint32(0x00200000)) & jnp.uint32(0xFFC00000), jnp.float32)
    q01 = jnp.where(a > 0.5, 1.0, 0.0) + jnp.where(a > 1.5, 1.0, 0.0)
    qa = jnp.where(a >= 2.0, bt, q01)
    q = lax.bitcast_convert_type(
        lax.bitcast_convert_type(qa, jnp.uint32) | sign, jnp.float32)
    o_ref[...] = scale * (q + 12.0) + mn


def kernel(x):
    orig_shape = x.shape
    xg = x.reshape(-1, GROUP)
    n = xg.shape[0]
    grid = (n // BLOCK_ROWS,)
    out = pl.pallas_call(
        _quant_body,
        grid=grid,
        in_specs=[pl.BlockSpec((BLOCK_ROWS, GROUP), lambda i: (i, 0))],
        out_specs=pl.BlockSpec((BLOCK_ROWS, GROUP), lambda i: (i, 0)),
        out_shape=jax.ShapeDtypeStruct((n, GROUP), jnp.float32),
    )(xg)
    return out.reshape(orig_shape)
